# baseline (speedup 1.0000x reference)
import functools

import jax
import jax.numpy as jnp
from jax import lax
from jax.experimental import pallas as pl
from jax.experimental.pallas import tpu as pltpu
from jax.experimental.pallas import tpu_sc as plsc

VOCAB = 100000
HIDDEN = 1024
NUM_CORES = 2
NUM_SUBCORES = 16
NW = NUM_CORES * NUM_SUBCORES

B_TOTAL = 8192
B_PER_W = B_TOTAL // NW
CHUNK = 32
CHUNKS = (CHUNK,) * (B_PER_W // CHUNK)
OFFS = tuple(i * CHUNK for i in range(len(CHUNKS)))
NCHUNK = len(CHUNKS)
NBUF = 2
SEG_PER_ROW = 2048 // B_PER_W


@jax.jit
def _embed_gather(idx, table):
    mesh = plsc.VectorSubcoreMesh(core_axis_name="c", subcore_axis_name="s")

    @functools.partial(
        pl.kernel,
        out_type=jax.ShapeDtypeStruct((B_TOTAL, HIDDEN), jnp.float32),
        mesh=mesh,
        scratch_types=[
            pltpu.VMEM((4, B_PER_W), jnp.int32),
            pltpu.VMEM((NBUF, CHUNK, HIDDEN), jnp.float32),
            pltpu.SemaphoreType.DMA,
            pltpu.SemaphoreType.DMA,
        ],
    )
    def k(table_hbm, idx_hbm, out_hbm, idx_v, rows_v, gsem, psem):
        wid = lax.axis_index("s") * NUM_CORES + lax.axis_index("c")
        base = wid * B_PER_W
        row = wid // SEG_PER_ROW
        col = pl.multiple_of((wid % SEG_PER_ROW) * B_PER_W, B_PER_W)

        def start_gather(c, b):
            n = CHUNKS[c]
            pltpu.async_copy(
                table_hbm.at[idx_v.at[row, pl.ds(OFFS[c], n)]],
                rows_v.at[b, pl.ds(0, n)], gsem)

        def wait_gather(c, b):
            n = CHUNKS[c]
            pltpu.make_async_copy(
                out_hbm.at[pl.ds(0, n)], rows_v.at[b, pl.ds(0, n)],
                gsem).wait()

        def start_put(c, b):
            n = CHUNKS[c]
            pltpu.async_copy(
                rows_v.at[b, pl.ds(0, n)],
                out_hbm.at[pl.ds(base + OFFS[c], n)], psem)

        def wait_put(c, b):
            n = CHUNKS[c]
            pltpu.make_async_copy(
                rows_v.at[b, pl.ds(0, n)], out_hbm.at[pl.ds(0, n)],
                psem).wait()

        half = B_PER_W // 2
        pltpu.sync_copy(idx_hbm.at[:, pl.ds(col, half)],
                        idx_v.at[:, pl.ds(0, half)])
        start_gather(0, 0)
        start_gather(1, 1)
        pltpu.sync_copy(
            idx_hbm.at[:, pl.ds(col + half, half)],
            idx_v.at[:, pl.ds(half, half)])

        for c in range(NCHUNK):
            b = c % NBUF
            wait_gather(c, b)
            start_put(c, b)
            if c + NBUF < NCHUNK:
                wait_put(c, b)
                start_gather(c + NBUF, b)
        for c in range(NCHUNK - NBUF, NCHUNK):
            wait_put(c, c % NBUF)

    return k(table, idx)


def kernel(inputs, table):
    out = _embed_gather(inputs.astype(jnp.int32), table)
    return out.reshape(inputs.shape[0], inputs.shape[1], HIDDEN)


# speedup vs baseline: 1.0131x; 1.0131x over previous
import functools

import jax
import jax.numpy as jnp
from jax import lax
from jax.experimental import pallas as pl
from jax.experimental.pallas import tpu as pltpu
from jax.experimental.pallas import tpu_sc as plsc

VOCAB = 100000
HIDDEN = 1024
NUM_CORES = 2
NUM_SUBCORES = 16
NW = NUM_CORES * NUM_SUBCORES

B_TOTAL = 8192
B_PER_W = B_TOTAL // NW
CHUNK = 32
NCHUNK = B_PER_W // CHUNK
NBUF = 2


@jax.jit
def _embed_gather(idx, table):
    mesh = plsc.VectorSubcoreMesh(core_axis_name="c", subcore_axis_name="s")

    @functools.partial(
        pl.kernel,
        out_type=jax.ShapeDtypeStruct((B_TOTAL, HIDDEN), jnp.float32),
        mesh=mesh,
        scratch_types=[
            pltpu.VMEM((B_PER_W,), jnp.int32),
            pltpu.VMEM((NBUF, CHUNK, HIDDEN), jnp.float32),
            pltpu.SemaphoreType.DMA,
            pltpu.SemaphoreType.DMA,
        ],
    )
    def k(table_hbm, idx_hbm, out_hbm, idx_v, rows_v, gsem, psem):
        wid = lax.axis_index("s") * NUM_CORES + lax.axis_index("c")
        base = wid * B_PER_W
        pltpu.sync_copy(idx_hbm.at[pl.ds(base, B_PER_W)], idx_v)

        def start_gather(c, b):
            pltpu.async_copy(
                table_hbm.at[idx_v.at[pl.ds(c * CHUNK, CHUNK)]],
                rows_v.at[b], gsem)

        def wait_gather(b):
            pltpu.make_async_copy(
                out_hbm.at[pl.ds(0, CHUNK)], rows_v.at[b], gsem).wait()

        def start_put(c, b):
            pltpu.async_copy(
                rows_v.at[b], out_hbm.at[pl.ds(base + c * CHUNK, CHUNK)],
                psem)

        def wait_put(b):
            pltpu.make_async_copy(
                rows_v.at[b], out_hbm.at[pl.ds(0, CHUNK)], psem).wait()

        for b in range(NBUF):
            start_gather(b, b)

        @pl.loop(0, NCHUNK - NBUF, step=NBUF)
        def _(c):
            for b in range(NBUF):
                cc = c + b
                wait_gather(b)
                start_put(cc, b)
                wait_put(b)
                start_gather(cc + NBUF, b)

        for b in range(NBUF):
            wait_gather(b)
            start_put(NCHUNK - NBUF + b, b)
        for b in range(NBUF):
            wait_put(b)

    return k(table, idx)


def kernel(inputs, table):
    idx = inputs.astype(jnp.int32).reshape(B_TOTAL)
    out = _embed_gather(idx, table)
    return out.reshape(inputs.shape[0], inputs.shape[1], HIDDEN)
